# baseline (speedup 1.0000x reference)
import functools

import jax
import jax.numpy as jnp
import numpy as np
from jax import lax
from jax.experimental import pallas as pl
from jax.experimental.pallas import tpu as pltpu
from jax.experimental.pallas import tpu_sc as plsc

NW = 32
GATHER_N = 128
CHUNK_ROWS = 1664


def _sc_gather(tabT, idxT, bsz, f, d):
    n_planes = f * d
    per_w = n_planes // NW
    schunk = 8192
    assert n_planes % NW == 0
    n_sch = bsz // schunk
    mesh = plsc.VectorSubcoreMesh(core_axis_name="c", subcore_axis_name="s")

    @functools.partial(
        pl.kernel, mesh=mesh,
        out_type=jax.ShapeDtypeStruct((n_planes, bsz), jnp.float32),
        compiler_params=pltpu.CompilerParams(
            use_tc_tiling_on_sc=False, needs_layout_passes=False),
        scratch_types=[
            pltpu.VMEM((tabT.shape[2],), jnp.float32),
            pltpu.VMEM((bsz,), jnp.int32),
            pltpu.VMEM((schunk,), jnp.float32),
            pltpu.SemaphoreType.DMA,
        ],
    )
    def k(tab_hbm, idx_hbm, out_hbm, plane_v, idx_v, out_v, sem):
        wid = lax.axis_index("s") * 2 + lax.axis_index("c")

        def plane_body(j, prev_fi):
            p = wid * per_w + j
            fi = p // d
            dd = p % d
            cp_plane = pltpu.async_copy(tab_hbm.at[fi, dd, :], plane_v, sem)

            @pl.when(fi != prev_fi)
            def _():
                pltpu.sync_copy(idx_hbm.at[fi, :], idx_v)

            cp_plane.wait()

            def chunk_body(c, carry2):
                s0 = c * schunk

                def g_body(g, carry3):
                    base = s0 + g * 64
                    for u in range(4):
                        sl = pl.ds(base + u * 16, 16)
                        osl = pl.ds(g * 64 + u * 16, 16)
                        out_v[osl] = plsc.load_gather(plane_v, [idx_v[sl]])
                    return carry3

                lax.fori_loop(0, schunk // 64, g_body, 0)
                pltpu.sync_copy(out_v, out_hbm.at[p, pl.ds(s0, schunk)])
                return carry2

            lax.fori_loop(0, n_sch, chunk_body, 0)
            return fi

        lax.fori_loop(0, per_w, plane_body, jnp.int32(-1))

    return k(tabT, idxT)


def _tc_mlp(ft2_parts, w1at, w1bft, b1c, w2t, b2c, w3t, b3c, bb, f, d):
    b = ft2_parts[0].shape[1]
    h1 = w1at.shape[0]
    h2 = w2t.shape[0]
    n_parts = len(ft2_parts)

    def body(*refs):
        part_refs = refs[:n_parts]
        (w1a_ref, w1b_ref, b1_ref, w2_ref, b2_ref, w3_ref, b3_ref,
         out_ref) = refs[n_parts:]
        flat_t = jnp.concatenate([r[...] for r in part_refs], axis=0)
        ft = flat_t.reshape(f, d, bb)
        gs = []
        for i in range(f):
            prod = ft * ft[i][None]
            gs.append(jnp.sum(prod, axis=1))
        gt = jnp.concatenate(gs, axis=0)
        bf = jnp.bfloat16
        h = jnp.dot(w1a_ref[...].astype(bf), flat_t.astype(bf),
                    preferred_element_type=jnp.float32)
        h = h + jnp.dot(w1b_ref[...].astype(bf), gt.astype(bf),
                        preferred_element_type=jnp.float32)
        h = jnp.maximum(h + b1_ref[...], 0.0)
        h = jnp.dot(w2_ref[...].astype(bf), h.astype(bf),
                    preferred_element_type=jnp.float32)
        h = jnp.maximum(h + b2_ref[...], 0.0)
        o = jnp.dot(w3_ref[...].astype(bf), h.astype(bf),
                    preferred_element_type=jnp.float32)
        out_ref[...] = o + b3_ref[...]

    return pl.pallas_call(
        body,
        grid=(b // bb,),
        in_specs=[
            pl.BlockSpec((p.shape[0], bb), lambda i: (0, i))
            for p in ft2_parts
        ] + [
            pl.BlockSpec((h1, f * d), lambda i: (0, 0)),
            pl.BlockSpec((h1, f * f), lambda i: (0, 0)),
            pl.BlockSpec((h1, 1), lambda i: (0, 0)),
            pl.BlockSpec((h2, h1), lambda i: (0, 0)),
            pl.BlockSpec((h2, 1), lambda i: (0, 0)),
            pl.BlockSpec((1, h2), lambda i: (0, 0)),
            pl.BlockSpec((1, 1), lambda i: (0, 0)),
        ],
        out_specs=pl.BlockSpec((1, bb), lambda i: (0, i)),
        out_shape=jax.ShapeDtypeStruct((1, b), jnp.float32),
    )(*ft2_parts, w1at, w1bft, b1c, w2t, b2c, w3t, b3c)


def kernel(indices, tables, W1, b1, W2, b2, W3, b3):
    bsz, f = indices.shape
    _, v, d = tables.shape
    h1 = W1.shape[1]

    tabT = jnp.transpose(tables, (0, 2, 1))
    idxT = indices.T
    groups = (26,)
    ft2_parts = []
    f0 = 0
    for fg in groups:
        ft2_parts.append(_sc_gather(tabT[f0:f0 + fg], idxT[f0:f0 + fg],
                                    bsz, fg, d))
        f0 += fg

    iu, ju = np.triu_indices(f, k=1)
    w1a = W1[: f * d]
    w1bf = jnp.zeros((f * f, h1), W1.dtype).at[iu * f + ju].set(W1[f * d:])

    out2 = _tc_mlp(ft2_parts, w1a.T, w1bf.T, b1[:, None], W2.T, b2[:, None],
                   W3.T, b3[None, :], 1024, f, d)
    return out2.reshape(bsz)


# speedup vs baseline: 1.0168x; 1.0168x over previous
import functools

import jax
import jax.numpy as jnp
import numpy as np
from jax import lax
from jax.experimental import pallas as pl
from jax.experimental.pallas import tpu as pltpu
from jax.experimental.pallas import tpu_sc as plsc

NW = 32
GATHER_N = 128
CHUNK_ROWS = 1664


def _sc_gather(tabT, idxT, bsz, f, d):
    n_planes = f * d
    per_w = n_planes // NW
    schunk = 8192
    assert n_planes % NW == 0
    n_sch = bsz // schunk
    mesh = plsc.VectorSubcoreMesh(core_axis_name="c", subcore_axis_name="s")

    @functools.partial(
        pl.kernel, mesh=mesh,
        out_type=jax.ShapeDtypeStruct((n_planes, bsz), jnp.float32),
        compiler_params=pltpu.CompilerParams(
            use_tc_tiling_on_sc=False, needs_layout_passes=False),
        scratch_types=[
            pltpu.VMEM((tabT.shape[2],), jnp.float32),
            pltpu.VMEM((bsz,), jnp.int32),
            pltpu.VMEM((schunk,), jnp.float32),
            pltpu.SemaphoreType.DMA,
        ],
    )
    def k(tab_hbm, idx_hbm, out_hbm, plane_v, idx_v, out_v, sem):
        wid = lax.axis_index("s") * 2 + lax.axis_index("c")

        def plane_body(j, prev_fi):
            p = wid * per_w + j
            fi = p // d
            dd = p % d
            cp_plane = pltpu.async_copy(tab_hbm.at[fi, dd, :], plane_v, sem)

            @pl.when(fi != prev_fi)
            def _():
                pltpu.sync_copy(idx_hbm.at[fi, :], idx_v)

            cp_plane.wait()

            def chunk_body(c, carry2):
                s0 = c * schunk

                def g_body(g, carry3):
                    base = s0 + g * 64
                    for u in range(4):
                        sl = pl.ds(base + u * 16, 16)
                        osl = pl.ds(g * 64 + u * 16, 16)
                        out_v[osl] = plsc.load_gather(plane_v, [idx_v[sl]])
                    return carry3

                lax.fori_loop(0, schunk // 64, g_body, 0)
                pltpu.sync_copy(out_v, out_hbm.at[p, pl.ds(s0, schunk)])
                return carry2

            lax.fori_loop(0, n_sch, chunk_body, 0)
            return fi

        lax.fori_loop(0, per_w, plane_body, jnp.int32(-1))

    return k(tabT, idxT)


def _tc_mlp(ft2_parts, w1at, w1bft, b1c, w2t, b2c, w3t, b3c, bb, f, d):
    b = ft2_parts[0].shape[1]
    h1 = w1at.shape[0]
    h2 = w2t.shape[0]
    n_parts = len(ft2_parts)

    def body(*refs):
        part_refs = refs[:n_parts]
        (w1a_ref, w1b_ref, b1_ref, w2_ref, b2_ref, w3_ref, b3_ref,
         out_ref) = refs[n_parts:]
        flat_t = jnp.concatenate([r[...] for r in part_refs], axis=0)
        ft = flat_t.reshape(f, d, bb)
        gs = []
        for i in range(f - 1):
            prod = ft[i + 1:] * ft[i][None]
            gs.append(jnp.sum(prod, axis=1))
        gt = jnp.concatenate(gs, axis=0)
        bf = jnp.bfloat16
        h = jnp.dot(w1a_ref[...].astype(bf), flat_t.astype(bf),
                    preferred_element_type=jnp.float32)
        h = h + jnp.dot(w1b_ref[...].astype(bf), gt.astype(bf),
                        preferred_element_type=jnp.float32)
        h = jnp.maximum(h + b1_ref[...], 0.0)
        h = jnp.dot(w2_ref[...].astype(bf), h.astype(bf),
                    preferred_element_type=jnp.float32)
        h = jnp.maximum(h + b2_ref[...], 0.0)
        o = jnp.dot(w3_ref[...].astype(bf), h.astype(bf),
                    preferred_element_type=jnp.float32)
        out_ref[...] = o + b3_ref[...]

    return pl.pallas_call(
        body,
        grid=(b // bb,),
        in_specs=[
            pl.BlockSpec((p.shape[0], bb), lambda i: (0, i))
            for p in ft2_parts
        ] + [
            pl.BlockSpec((h1, f * d), lambda i: (0, 0)),
            pl.BlockSpec((h1, f * (f - 1) // 2), lambda i: (0, 0)),
            pl.BlockSpec((h1, 1), lambda i: (0, 0)),
            pl.BlockSpec((h2, h1), lambda i: (0, 0)),
            pl.BlockSpec((h2, 1), lambda i: (0, 0)),
            pl.BlockSpec((1, h2), lambda i: (0, 0)),
            pl.BlockSpec((1, 1), lambda i: (0, 0)),
        ],
        out_specs=pl.BlockSpec((1, bb), lambda i: (0, i)),
        out_shape=jax.ShapeDtypeStruct((1, b), jnp.float32),
    )(*ft2_parts, w1at, w1bft, b1c, w2t, b2c, w3t, b3c)


def kernel(indices, tables, W1, b1, W2, b2, W3, b3):
    bsz, f = indices.shape
    _, v, d = tables.shape
    h1 = W1.shape[1]

    tabT = jnp.transpose(tables, (0, 2, 1))
    idxT = indices.T
    groups = (26,)
    ft2_parts = []
    f0 = 0
    for fg in groups:
        ft2_parts.append(_sc_gather(tabT[f0:f0 + fg], idxT[f0:f0 + fg],
                                    bsz, fg, d))
        f0 += fg

    w1a = W1[: f * d]
    w1b = W1[f * d:]

    out2 = _tc_mlp(ft2_parts, w1a.T, w1b.T, b1[:, None], W2.T, b2[:, None],
                   W3.T, b3[None, :], 1024, f, d)
    return out2.reshape(bsz)
